# Initial kernel scaffold; baseline (speedup 1.0000x reference)
import jax
import jax.numpy as jnp
from jax.experimental import pallas as pl


def kernel(hidden_states, W_gate, mup_W, gW, g_qa, g_qb, g_sc, uW, u_qa, u_qb, u_sc, dW, d_qa, d_qb, d_sc, sh_gW, sh_uW, sh_dW, inv_mapping):
    raise NotImplementedError("write your pallas kernel here")


# speedup vs baseline: 2.0100x; 2.0100x over previous
import functools

import jax
import jax.numpy as jnp
from jax import lax
from jax.experimental import pallas as pl
from jax.experimental.pallas import tpu as pltpu

_NE = 64
_NF = 16
_NPF = 4
_TOPK = 8
_R = 8


def _dotg(a, b, dims, out_dtype=jnp.float32):
    return lax.dot_general(a, b, (dims, ((), ())),
                           preferred_element_type=out_dtype)


def _gate_shared_kernel(h_ref, wg_ref, sgw_ref, suw_ref, sdw_ref,
                        fw_ref, sh_ref):
    h = h_ref[...]
    logits = _dotg(h, wg_ref[...], ((1,), (1,)))
    s = jax.nn.softmax(logits, axis=-1)
    ii = lax.broadcasted_iota(jnp.int32, s.shape, 1)
    cur = s
    selw = jnp.zeros_like(s)
    for _ in range(_TOPK):
        m = jnp.max(cur, axis=-1, keepdims=True)
        cand = jnp.where(cur == m, ii, _NE)
        j = jnp.min(cand, axis=-1, keepdims=True)
        hit = ii == j
        selw = jnp.where(hit, s, selw)
        cur = jnp.where(hit, -1.0, cur)
    fw_ref[...] = selw
    hb = h.astype(jnp.bfloat16)
    g = _dotg(hb, sgw_ref[...].astype(jnp.bfloat16), ((1,), (1,)))
    u = _dotg(hb, suw_ref[...].astype(jnp.bfloat16), ((1,), (1,)))
    gu = (jax.nn.silu(g) * u).astype(jnp.bfloat16)
    sh = _dotg(gu, sdw_ref[...].astype(jnp.bfloat16), ((1,), (1,)))
    sh_ref[...] = sh.astype(jnp.bfloat16)


def _moe_kernel(hb_ref, fw_ref, shb_ref,
                gw_ref, gqa_ref, gqb_ref, gsc_ref,
                uw_ref, uqa_ref, uqb_ref, usc_ref,
                dw_ref, dqa_ref, dqb_ref, dsc_ref,
                mup_ref, y_ref, *, bt):
    e = pl.program_id(0)
    t = pl.program_id(1)
    rows = pl.ds(t * bt, bt)

    fw_all = fw_ref[rows, :]
    ri = lax.broadcasted_iota(jnp.int32, (_NE, _NPF), 0)
    ci = lax.broadcasted_iota(jnp.int32, (_NE, _NPF), 1)
    sel = (ri == ci + e * _NPF).astype(jnp.float32)
    fw_raw = _dotg(fw_all, sel, ((1,), (0,)))
    scalar = jnp.sum(fw_raw, axis=-1, keepdims=True)
    fwx = jax.nn.softmax(
        jnp.where(fw_raw == 0.0, -1e9, fw_raw), axis=-1)
    gi = lax.broadcasted_iota(jnp.int32, (_NPF, _NPF * _R), 0)
    li = lax.broadcasted_iota(jnp.int32, (_NPF, _NPF * _R), 1)
    rep = (gi == li // _R).astype(jnp.float32)
    fw32 = _dotg(fwx, rep, ((1,), (0,)))

    x = hb_ref[rows, :].astype(jnp.float32) + _dotg(fwx, mup_ref[0],
                                                    ((1,), (0,)))
    xb = x.astype(jnp.bfloat16)

    def fused(inp_b, w_ref, qa_ref, qb_ref, sc_ref):
        main = _dotg(inp_b, w_ref[0], ((1,), (1,)))
        t1 = _dotg(inp_b, qa_ref[0], ((1,), (1,)))
        t1 = (t1 * fw32).astype(jnp.bfloat16)
        t2 = _dotg(t1, qb_ref[0], ((1,), (0,)))
        return main + sc_ref[0] * t2

    g = jax.nn.silu(fused(xb, gw_ref, gqa_ref, gqb_ref, gsc_ref))
    u = fused(xb, uw_ref, uqa_ref, uqb_ref, usc_ref)
    gu = (g * u).astype(jnp.bfloat16)
    d = fused(gu, dw_ref, dqa_ref, dqb_ref, dsc_ref)
    contrib = scalar * d

    @pl.when(e == 0)
    def _():
        y_ref[rows, :] = shb_ref[rows, :].astype(jnp.float32) + contrib

    @pl.when(e > 0)
    def _():
        y_ref[rows, :] += contrib


@functools.partial(jax.jit, static_argnames=("interpret",))
def _run(hidden_states, W_gate, mup_W, gW, g_qa, g_qb, g_sc,
         uW, u_qa, u_qb, u_sc, dW, d_qa, d_qb, d_sc,
         sh_gW, sh_uW, sh_dW, interpret=False):
    orig_shape = hidden_states.shape
    H = orig_shape[-1]
    h = hidden_states.reshape(-1, H)
    T = h.shape[0]
    I = gW.shape[1]
    bf = jnp.bfloat16

    hb = h.astype(bf)
    gqa = g_qa.reshape(_NF, _NPF * _R, H).astype(bf)
    uqa = u_qa.reshape(_NF, _NPF * _R, H).astype(bf)
    dqa = d_qa.reshape(_NF, _NPF * _R, I).astype(bf)
    gqb = g_qb.transpose(0, 1, 3, 2).reshape(_NF, _NPF * _R, I).astype(bf)
    uqb = u_qb.transpose(0, 1, 3, 2).reshape(_NF, _NPF * _R, I).astype(bf)
    dqb = d_qb.transpose(0, 1, 3, 2).reshape(_NF, _NPF * _R, H).astype(bf)
    mupT = mup_W.transpose(0, 2, 1)
    gsc = g_sc.reshape(_NF, 1, I)
    usc = u_sc.reshape(_NF, 1, I)
    dsc = d_sc.reshape(_NF, 1, H)
    gWb = gW.astype(bf)
    uWb = uW.astype(bf)
    dWb = dW.astype(bf)

    bta = min(512, T)
    nta = T // bta
    fw, shb = pl.pallas_call(
        _gate_shared_kernel,
        grid=(nta,),
        in_specs=[
            pl.BlockSpec((bta, H), lambda t: (t, 0)),
            pl.BlockSpec((_NE, H), lambda t: (0, 0)),
            pl.BlockSpec(sh_gW.shape, lambda t: (0, 0)),
            pl.BlockSpec(sh_uW.shape, lambda t: (0, 0)),
            pl.BlockSpec(sh_dW.shape, lambda t: (0, 0)),
        ],
        out_specs=[
            pl.BlockSpec((bta, _NE), lambda t: (t, 0)),
            pl.BlockSpec((bta, H), lambda t: (t, 0)),
        ],
        out_shape=[
            jax.ShapeDtypeStruct((T, _NE), jnp.float32),
            jax.ShapeDtypeStruct((T, H), bf),
        ],
        interpret=interpret,
    )(h, W_gate, sh_gW, sh_uW, sh_dW)

    bt = min(512, T)
    nt = T // bt
    res = lambda e, t: (0, 0)
    per_e = lambda e, t: (e, 0, 0)
    y = pl.pallas_call(
        functools.partial(_moe_kernel, bt=bt),
        grid=(_NF, nt),
        in_specs=[
            pl.BlockSpec((T, H), res),
            pl.BlockSpec((T, _NE), res),
            pl.BlockSpec((T, H), res),
            pl.BlockSpec((1, I, H), per_e),
            pl.BlockSpec((1, _NPF * _R, H), per_e),
            pl.BlockSpec((1, _NPF * _R, I), per_e),
            pl.BlockSpec((1, 1, I), per_e),
            pl.BlockSpec((1, I, H), per_e),
            pl.BlockSpec((1, _NPF * _R, H), per_e),
            pl.BlockSpec((1, _NPF * _R, I), per_e),
            pl.BlockSpec((1, 1, I), per_e),
            pl.BlockSpec((1, H, I), per_e),
            pl.BlockSpec((1, _NPF * _R, I), per_e),
            pl.BlockSpec((1, _NPF * _R, H), per_e),
            pl.BlockSpec((1, 1, H), per_e),
            pl.BlockSpec((1, _NPF, H), per_e),
        ],
        out_specs=pl.BlockSpec((T, H), res),
        out_shape=jax.ShapeDtypeStruct((T, H), jnp.float32),
        interpret=interpret,
    )(hb, fw, shb,
      gWb, gqa, gqb, gsc,
      uWb, uqa, uqb, usc,
      dWb, dqa, dqb, dsc,
      mupT)

    return y.reshape(orig_shape)


def kernel(hidden_states, W_gate, mup_W, gW, g_qa, g_qb, g_sc,
           uW, u_qa, u_qb, u_sc, dW, d_qa, d_qb, d_sc,
           sh_gW, sh_uW, sh_dW, inv_mapping):
    del inv_mapping
    return _run(hidden_states, W_gate, mup_W, gW, g_qa, g_qb, g_sc,
                uW, u_qa, u_qb, u_sc, dW, d_qa, d_qb, d_sc,
                sh_gW, sh_uW, sh_dW)
